# baseline (speedup 1.0000x reference)
import jax
import jax.numpy as jnp
from jax import lax
from jax.experimental import pallas as pl
from jax.experimental.pallas import tpu as pltpu
from jax.experimental.pallas import tpu_sc as plsc

_B = 16384
_NB1 = 16
_ED = 8
_TAB = 33 * _ED
_STR = 17
_TABP = 576
_W = 4096
_ROW = 131072
_NU = 48


def _body(x_hbm, tab_hbm, out_hbm, tab_v, tabs_v, x_v, out_v, sem_x, sem_o):
    nc = 2
    wid = lax.axis_index("s") * nc + lax.axis_index("c")
    b1 = wid // 2
    hh = wid % 2
    pltpu.sync_copy(tab_hbm, tab_v)
    lane = lax.iota(jnp.int32, 16)
    for i in range(_TABP // 16):
        pos = i * 16 + lane
        src = pos // _STR * _ED + jnp.minimum(pos % _STR, _ED - 1)
        row = plsc.load_gather(tab_v, [jnp.minimum(src, _TAB - 1)])
        tabs_v[pl.ds(i * 16, 16)] = row

    def xrow_off(rv):
        c = rv // 8
        kk = (rv // 2) % 4
        bh = rv % 2
        hc = (hh * 4 + kk) * 3 + c
        return hc * _ROW + b1 * 8192 + bh * _W

    pltpu.async_copy(x_hbm.at[pl.ds(xrow_off(0), _W)], x_v.at[0], sem_x)

    def unit(u, _):
        c = u // 16
        k = (u // 4) % 4
        bh = (u // 2) % 2
        dq = u % 2
        p = u % 4
        rv = u // 2
        slot = rv % 4

        @pl.when(dq == 0)
        def _x_ring():
            pltpu.make_async_copy(
                x_hbm.at[pl.ds(0, _W)], x_v.at[0], sem_x
            ).wait()

            @pl.when(rv + 1 < 24)
            def _prefetch():
                pltpu.async_copy(
                    x_hbm.at[pl.ds(xrow_off(rv + 1), _W)],
                    x_v.at[(rv + 1) % 4],
                    sem_x,
                )

        @pl.when(u >= 4)
        def _wait_drain():
            pltpu.make_async_copy(
                out_hbm.at[pl.ds(0, 4 * _W)], out_v.at[p], sem_o.at[p]
            ).wait()

        base = 11 * _STR * c + 4 * dq
        tds = [jnp.broadcast_to(base + i, (16,)).astype(jnp.int32) for i in range(4)]

        @plsc.parallel_loop(0, _W // 16, 1, unroll=8)
        def j_body(j):
            o = j * 16
            x17 = x_v[slot, pl.ds(o, 16)] * _STR
            for i in range(4):
                ev = plsc.load_gather(tabs_v, [x17 + tds[i]])
                out_v[p, pl.ds(i * _W + o, 16)] = ev

        r0 = (c * _ED + 4 * dq) * 8 + hh * 4 + k
        for i in range(4):
            pltpu.async_copy(
                out_v.at[p, pl.ds(i * _W, _W)],
                out_hbm.at[pl.ds((r0 + i * 8) * _ROW + b1 * 8192 + bh * _W, _W)],
                sem_o.at[p],
            )
        return 0

    lax.fori_loop(0, _NU, unit, 0)
    for pp in range(4):
        pltpu.make_async_copy(
            out_hbm.at[pl.ds(0, 4 * _W)], out_v.at[pp], sem_o.at[pp]
        ).wait()


@jax.jit
def kernel(x, table):
    x1 = (
        x.reshape(_NB1, 8, 128, 8, 8, 3)
        .transpose(3, 5, 0, 1, 4, 2)
        .reshape(-1)
    )
    tab_flat = table.reshape(-1)
    mesh = plsc.VectorSubcoreMesh(core_axis_name="c", subcore_axis_name="s")
    out1 = pl.kernel(
        _body,
        out_type=jax.ShapeDtypeStruct((192 * _ROW,), jnp.float32),
        mesh=mesh,
        compiler_params=pltpu.CompilerParams(needs_layout_passes=False),
        scratch_types=[
            pltpu.VMEM((_TAB,), jnp.float32),
            pltpu.VMEM((_TABP,), jnp.float32),
            pltpu.VMEM((4, _W), jnp.int32),
            pltpu.VMEM((4, 4 * _W), jnp.float32),
            pltpu.SemaphoreType.DMA,
            pltpu.SemaphoreType.DMA((4,)),
        ],
    )(x1, tab_flat)
    return (
        out1.reshape(24, 8, _NB1, 8, 8, 128)
        .transpose(2, 3, 5, 0, 1, 4)
        .reshape(_B, 24, 8, 8)
    )
